# baseline (speedup 1.0000x reference)
import functools

import jax
import jax.numpy as jnp
from jax import lax
from jax.experimental import pallas as pl
from jax.experimental.pallas import tpu as pltpu
from jax.experimental.pallas import tpu_sc as plsc

_B = 16384
_L = 200
_D = 64
VOCAB_ = 1000000
_S0 = 128
_S1 = _L - _S0
_NC = 2
_NS = 16
_NW = _NC * _NS
_BPT = _B // _NW
_CK = 16


def _sc_partial_sums(x2, table):
    mesh = plsc.VectorSubcoreMesh(core_axis_name="c", subcore_axis_name="s")

    @functools.partial(
        pl.kernel,
        out_type=jax.ShapeDtypeStruct((_NW, _L, _D), jnp.float32),
        mesh=mesh,
        scratch_types=[
            pltpu.VMEM((_BPT, _L), jnp.int32),
            pltpu.VMEM((_L, _D), jnp.float32),
            pltpu.SemaphoreType.DMA,
        ],
        compiler_params=pltpu.CompilerParams(use_tc_tiling_on_sc=False),
    )
    def k(x_hbm, tab_hbm, out_hbm, idx_v, acc_v, sem):
        wid = lax.axis_index("s") * _NC + lax.axis_index("c")
        base = wid * _BPT
        pltpu.sync_copy(x_hbm.at[pl.ds(base, _BPT)], idx_v)

        zeros = jnp.zeros((16,), jnp.float32)

        def zbody(l, carry):
            for d in range(_D // 16):
                acc_v[l, pl.ds(16 * d, 16)] = zeros
            return carry

        lax.fori_loop(0, _L, zbody, 0)

        acc_lo = acc_v.at[pl.ds(0, _S0)]
        acc_hi = acc_v.at[pl.ds(_S0, _S1)]

        def fire_chunk(c):
            descs = []
            for r in range(_CK):
                j = c * _CK + r
                descs.append(pltpu.async_copy(
                    tab_hbm.at[idx_v.at[j, pl.ds(0, _S0)]], acc_lo, sem,
                    add=True))
                descs.append(pltpu.async_copy(
                    tab_hbm.at[idx_v.at[j, pl.ds(_S0, _S1)]], acc_hi, sem,
                    add=True))
            return descs

        fire_chunk(0)

        def body(c, carry):
            descs = fire_chunk(c)
            for d in descs:
                d.wait()
            return carry

        nchunks = _BPT // _CK
        lax.fori_loop(1, nchunks, body, 0)
        for r in range(_CK):
            pltpu.make_async_copy(
                tab_hbm.at[idx_v.at[0, pl.ds(0, _S0)]], acc_lo, sem).wait()
            pltpu.make_async_copy(
                tab_hbm.at[idx_v.at[0, pl.ds(_S0, _S1)]], acc_hi, sem).wait()
        pltpu.sync_copy(acc_v, out_hbm.at[wid])

    return k(x2, table)


_TB = 32768
_NTB = (VOCAB_ + _TB - 1) // _TB
_NP2 = _NTB * (_TB // 2) + (VOCAB_ - (_NTB - 1) * _TB) - (_TB // 2)
_V2 = 2 * _NP2


def _tc_relayout(wt):
    def body(w_ref, o_ref):
        y = w_ref[...].T
        o_ref[:, 0:_D] = y[0:_TB // 2]
        o_ref[:, _D:128] = y[_TB // 2:_TB]

    return pl.pallas_call(
        body,
        grid=(_NTB,),
        in_specs=[pl.BlockSpec((_D, _TB), lambda j: (0, j))],
        out_specs=pl.BlockSpec((_TB // 2, 128), lambda j: (j, 0)),
        out_shape=jax.ShapeDtypeStruct((_NP2, 128), jnp.float32),
    )(wt)


def _remap_idx(x):
    r = jnp.bitwise_and(x, _TB - 1)
    return x + r - jnp.where(r >= _TB // 2, _TB - 1, 0)


def _tc_reduce(partials):
    def body(p_ref, o_ref):
        o_ref[...] = jnp.sum(p_ref[...], axis=0) * (1.0 / _B)

    return pl.pallas_call(
        body,
        out_shape=jax.ShapeDtypeStruct((_L, _D), jnp.float32),
    )(partials)


def kernel(x, wordvec_weights):
    x2 = _remap_idx(x.astype(jnp.int32))
    table_lin = _tc_relayout(wordvec_weights.T).reshape(_V2, _D)
    partials = _sc_partial_sums(x2, table_lin)
    return _tc_reduce(partials)


# speedup vs baseline: 1.0052x; 1.0052x over previous
import functools

import jax
import jax.numpy as jnp
from jax import lax
from jax.experimental import pallas as pl
from jax.experimental.pallas import tpu as pltpu
from jax.experimental.pallas import tpu_sc as plsc

_B = 16384
_L = 200
_D = 64
VOCAB_ = 1000000
_S0 = 128
_S1 = _L - _S0
_NC = 2
_NS = 16
_NW = _NC * _NS
_BPT = _B // _NW
_CK = 8


def _sc_partial_sums(x2, table):
    mesh = plsc.VectorSubcoreMesh(core_axis_name="c", subcore_axis_name="s")

    @functools.partial(
        pl.kernel,
        out_type=jax.ShapeDtypeStruct((_NW, _L, _D), jnp.float32),
        mesh=mesh,
        scratch_types=[
            pltpu.VMEM((_BPT, _L), jnp.int32),
            pltpu.VMEM((_L, _D), jnp.float32),
            pltpu.SemaphoreType.DMA,
        ],
        compiler_params=pltpu.CompilerParams(use_tc_tiling_on_sc=False),
    )
    def k(x_hbm, tab_hbm, out_hbm, idx_v, acc_v, sem):
        wid = lax.axis_index("s") * _NC + lax.axis_index("c")
        base = wid * _BPT
        stage = pltpu.async_copy(x_hbm.at[pl.ds(base, _BPT)], idx_v, sem)

        zeros = jnp.zeros((16,), jnp.float32)

        def zbody(l, carry):
            for d in range(_D // 16):
                acc_v[l, pl.ds(16 * d, 16)] = zeros
            return carry

        lax.fori_loop(0, _L, zbody, 0)
        stage.wait()

        acc_lo = acc_v.at[pl.ds(0, _S0)]
        acc_hi = acc_v.at[pl.ds(_S0, _S1)]

        def fire_chunk(c):
            descs = []
            for r in range(_CK):
                j = c * _CK + r
                descs.append(pltpu.async_copy(
                    tab_hbm.at[idx_v.at[j, pl.ds(0, _S0)]], acc_lo, sem,
                    add=True))
                descs.append(pltpu.async_copy(
                    tab_hbm.at[idx_v.at[j, pl.ds(_S0, _S1)]], acc_hi, sem,
                    add=True))
            return descs

        fire_chunk(0)

        def body(c, carry):
            descs = fire_chunk(c)
            for d in descs:
                d.wait()
            return carry

        nchunks = _BPT // _CK
        lax.fori_loop(1, nchunks, body, 0)
        for r in range(_CK):
            pltpu.make_async_copy(
                tab_hbm.at[idx_v.at[0, pl.ds(0, _S0)]], acc_lo, sem).wait()
            pltpu.make_async_copy(
                tab_hbm.at[idx_v.at[0, pl.ds(_S0, _S1)]], acc_hi, sem).wait()
        pltpu.sync_copy(acc_v, out_hbm.at[wid])

    return k(x2, table)


_TB = 32768
_NTB = (VOCAB_ + _TB - 1) // _TB
_NP2 = _NTB * (_TB // 2) + (VOCAB_ - (_NTB - 1) * _TB) - (_TB // 2)
_V2 = 2 * _NP2


def _tc_relayout(wt):
    def body(w_ref, o_ref):
        y = w_ref[...].T
        o_ref[:, 0:_D] = y[0:_TB // 2]
        o_ref[:, _D:128] = y[_TB // 2:_TB]

    return pl.pallas_call(
        body,
        grid=(_NTB,),
        in_specs=[pl.BlockSpec((_D, _TB), lambda j: (0, j))],
        out_specs=pl.BlockSpec((_TB // 2, 128), lambda j: (j, 0)),
        out_shape=jax.ShapeDtypeStruct((_NP2, 128), jnp.float32),
    )(wt)


def _remap_idx(x):
    r = jnp.bitwise_and(x, _TB - 1)
    return x + r - jnp.where(r >= _TB // 2, _TB - 1, 0)


def _tc_reduce(partials):
    def body(p_ref, o_ref):
        o_ref[...] = jnp.sum(p_ref[...], axis=0) * (1.0 / _B)

    return pl.pallas_call(
        body,
        out_shape=jax.ShapeDtypeStruct((_L, _D), jnp.float32),
    )(partials)


def kernel(x, wordvec_weights):
    x2 = _remap_idx(x.astype(jnp.int32))
    table_lin = _tc_relayout(wordvec_weights.T).reshape(_V2, _D)
    partials = _sc_partial_sums(x2, table_lin)
    return _tc_reduce(partials)
